# Initial kernel scaffold; baseline (speedup 1.0000x reference)
import jax
import jax.numpy as jnp
from jax.experimental import pallas as pl


def kernel(ptr_t, a_list, v_list, t_embed, v_embed, a_embed, v_index, v_id_send, wv, wa_v, w1, w11, w12):
    raise NotImplementedError("write your pallas kernel here")


# speedup vs baseline: 59.4553x; 59.4553x over previous
import functools

import jax
import jax.numpy as jnp
from jax import lax
from jax.experimental import pallas as pl
from jax.experimental.pallas import tpu as pltpu
from jax.experimental.pallas import tpu_sc as plsc

N_T = 10000
N_V = 10000
E = 320000
D = 128
R = 5000
S = 5000
NPTR = N_T + 1

NTILES = 16
EPT = E // NTILES
BLK = 2000
NBLK = EPT // BLK
CH = 80
NCH = BLK // CH

ROWS_PT = 640
AGG_ROWS = NTILES * ROWS_PT
CTX_ROW = N_T
DUMP_ROW = N_T + 8

CTX_STRIDE = 312
CTX_LOAD = 320
CTX_PAD = 400


def _seg_search(ptr_v, seg_ref, ebase, chunk_base):
    for j in range(CH // 16):
        eg = ebase + chunk_base + j * 16 + lax.broadcasted_iota(jnp.int32, (16,), 0)
        lo = jnp.zeros((16,), jnp.int32)
        hi = jnp.full((16,), NPTR, jnp.int32)
        for _ in range(14):
            mid = jnp.right_shift(lo + hi, 1)
            pv = plsc.load_gather(ptr_v, [mid])
            cond = pv <= eg
            lo = jnp.where(cond, mid + 1, lo)
            hi = jnp.where(cond, hi, mid)
        seg_ref[0, pl.ds(j * 16, 16)] = jnp.clip(lo - 1, 0, N_T - 1)


def _sc_agg_build():
    mesh = plsc.VectorSubcoreMesh(core_axis_name="c", subcore_axis_name="s")

    @functools.partial(
        pl.kernel,
        mesh=mesh,
        compiler_params=pltpu.CompilerParams(needs_layout_passes=False),
        out_type=[
            jax.ShapeDtypeStruct((AGG_ROWS, D), jnp.float32),
            jax.ShapeDtypeStruct((AGG_ROWS, D), jnp.float32),
        ],
        scratch_types=[
            pltpu.VMEM((NPTR,), jnp.int32),
            pltpu.VMEM((BLK,), jnp.int32),
            pltpu.VMEM((1, CH), jnp.int32),
            pltpu.VMEM((CH, D), jnp.float32),
            pltpu.VMEM((CTX_PAD,), jnp.int32),
            pltpu.VMEM((CTX_PAD,), jnp.int32),
            pltpu.VMEM((S,), jnp.int32),
            pltpu.VMEM((CTX_PAD // CH, CH), jnp.int32),
            pltpu.VMEM_SHARED((AGG_ROWS, D), jnp.float32),
            pltpu.SemaphoreType.DMA,
        ],
    )
    def sc_agg(ptr_hbm, vlist_hbm, alist_hbm, vembed_hbm, aembed_hbm,
               vindex_hbm, vsend_hbm, aggv_hbm, agga_hbm,
               ptr_v, idxb, seg_ch, rows0, vind_b, i2_b, vsend_buf,
               ctx_seg, agg_sh, gsem):
        cid = lax.axis_index("c")
        sid = lax.axis_index("s")
        ebase = sid * EPT

        pltpu.sync_copy(ptr_hbm, ptr_v)

        def _zrow(i, c):
            for j in range(D // 16):
                rows0[i, pl.ds(j * 16, 16)] = jnp.zeros((16,), jnp.float32)
            return c
        lax.fori_loop(0, CH, _zrow, 0)
        for k in range(ROWS_PT // CH):
            pltpu.sync_copy(rows0, agg_sh.at[pl.ds(sid * ROWS_PT + k * CH, CH)])
        plsc.subcore_barrier()

        def _edge_loop(list_hbm, table_hbm):
            def blk(b, carry):
                pltpu.sync_copy(list_hbm.at[pl.ds(ebase + b * BLK, BLK)], idxb)

                def chunk(k, c2):
                    _seg_search(ptr_v, seg_ch, ebase, b * BLK + k * CH)
                    pltpu.async_copy(
                        table_hbm.at[idxb.at[pl.ds(k * CH, CH)]], rows0,
                        gsem).wait()
                    pltpu.sync_copy(rows0, agg_sh.at[seg_ch.at[0]], add=True)
                    return c2
                lax.fori_loop(0, NCH, chunk, carry)
                return carry
            lax.fori_loop(0, NBLK, blk, 0)

        @pl.when(cid == 0)
        def _():
            _edge_loop(vlist_hbm, vembed_hbm)

        @pl.when(cid == 1)
        def _():
            _edge_loop(alist_hbm, aembed_hbm)

        @pl.when(cid == 0)
        def _():
            pltpu.sync_copy(vsend_hbm, vsend_buf)
            pltpu.sync_copy(vindex_hbm.at[pl.ds(sid * CTX_STRIDE, CTX_LOAD)],
                            vind_b.at[pl.ds(0, CTX_LOAD)])
            for j in range((CTX_PAD - CTX_LOAD) // 16):
                vind_b[pl.ds(CTX_LOAD + j * 16, 16)] = jnp.zeros((16,), jnp.int32)
            cnt = jnp.where(sid == NTILES - 1, CTX_LOAD, CTX_STRIDE)
            for j in range(CTX_PAD // 16):
                l = j * 16 + lax.broadcasted_iota(jnp.int32, (16,), 0)
                msk = l < cnt
                vi = plsc.load_gather(vind_b, [jnp.where(msk, l, 0)])
                i2 = plsc.load_gather(vsend_buf, [vi])
                i2_b[pl.ds(j * 16, 16)] = i2
                jc, jr = divmod(j * 16, CH)
                ctx_seg[jc, pl.ds(jr, 16)] = jnp.where(msk, CTX_ROW, DUMP_ROW)
            for k in range(CTX_PAD // CH):
                pltpu.async_copy(vembed_hbm.at[i2_b.at[pl.ds(k * CH, CH)]],
                                 rows0, gsem).wait()
                pltpu.sync_copy(rows0, agg_sh.at[ctx_seg.at[k]], add=True)

        plsc.subcore_barrier()

        @pl.when(cid == 0)
        def _():
            pltpu.sync_copy(agg_sh.at[pl.ds(sid * ROWS_PT, ROWS_PT)],
                            aggv_hbm.at[pl.ds(sid * ROWS_PT, ROWS_PT)])

        @pl.when(cid == 1)
        def _():
            pltpu.sync_copy(agg_sh.at[pl.ds(sid * ROWS_PT, ROWS_PT)],
                            agga_hbm.at[pl.ds(sid * ROWS_PT, ROWS_PT)])

    return sc_agg


_sc_agg = _sc_agg_build()


def _mmT(a, b):
    return lax.dot_general(a, b, (((1,), (1,)), ((), ())),
                           precision=lax.Precision.HIGHEST,
                           preferred_element_type=jnp.float32)


def _mm(a, b):
    return lax.dot_general(a, b, (((1,), (0,)), ((), ())),
                           precision=lax.Precision.HIGHEST,
                           preferred_element_type=jnp.float32)


def _tc_body(t_ref, aggv_ref, agga_ref, wv_ref, wav_ref, w1_ref, w11_ref,
             w12_ref, out_ref):
    w1a = w1_ref[:, :D]
    w1b = w1_ref[:, D:]
    b1 = _mm(w1a, w11_ref[...])
    w1b12 = _mm(w1b, w12_ref[...])
    b2 = _mm(w1b12, wv_ref[...])
    b3 = _mm(w1b12, wav_ref[...])
    b4 = _mm(w1b, wv_ref[...])
    ctx = aggv_ref[CTX_ROW:CTX_ROW + 1, :] * (1.0 / R)
    ctxc = _mmT(ctx, b4)
    nb, bs = 10, N_T // 10
    for i in range(nb):
        sl = pl.ds(i * bs, bs)
        out_ref[sl, :] = (_mmT(t_ref[sl, :], b1) + _mmT(aggv_ref[sl, :], b2)
                          + _mmT(agga_ref[sl, :], b3) + ctxc)


def kernel(ptr_t, a_list, v_list, t_embed, v_embed, a_embed, v_index,
           v_id_send, wv, wa_v, w1, w11, w12):
    aggv, agga = _sc_agg(ptr_t, v_list, a_list, v_embed, a_embed,
                         v_index, v_id_send)
    out = pl.pallas_call(
        _tc_body,
        out_shape=jax.ShapeDtypeStruct((N_T, D), jnp.float32),
    )(t_embed, aggv, agga, wv, wa_v, w1, w11, w12)
    return out


# speedup vs baseline: 96.4309x; 1.6219x over previous
import functools

import jax
import jax.numpy as jnp
from jax import lax
from jax.experimental import pallas as pl
from jax.experimental.pallas import tpu as pltpu
from jax.experimental.pallas import tpu_sc as plsc

N_T = 10000
N_V = 10000
E = 320000
D = 128
R = 5000
S = 5000
NPTR = N_T + 1

NTILES = 16
EPT = E // NTILES
BLK = 2000
NBLK = EPT // BLK
CH = 80
NCH = BLK // CH

ROWS_PT = 640
AGG_ROWS = NTILES * ROWS_PT
CTX_ROW = N_T
DUMP_ROW = N_T + 8

CTX_STRIDE = 312
CTX_LOAD = 320
CTX_PAD = 400


def _seg_search(ptr_v, seg_ref, ebase, chunk_base):
    for j in range(CH // 16):
        eg = ebase + chunk_base + j * 16 + lax.broadcasted_iota(jnp.int32, (16,), 0)
        lo = jnp.zeros((16,), jnp.int32)
        hi = jnp.full((16,), NPTR, jnp.int32)
        for _ in range(14):
            mid = jnp.right_shift(lo + hi, 1)
            pv = plsc.load_gather(ptr_v, [mid])
            cond = pv <= eg
            lo = jnp.where(cond, mid + 1, lo)
            hi = jnp.where(cond, hi, mid)
        seg_ref[0, pl.ds(j * 16, 16)] = jnp.clip(lo - 1, 0, N_T - 1)


def _sc_agg_build():
    mesh = plsc.VectorSubcoreMesh(core_axis_name="c", subcore_axis_name="s")

    @functools.partial(
        pl.kernel,
        mesh=mesh,
        compiler_params=pltpu.CompilerParams(needs_layout_passes=False),
        out_type=[
            jax.ShapeDtypeStruct((AGG_ROWS, D), jnp.float32),
            jax.ShapeDtypeStruct((AGG_ROWS, D), jnp.float32),
        ],
        scratch_types=[
            pltpu.VMEM((NPTR,), jnp.int32),
            pltpu.VMEM((BLK,), jnp.int32),
            pltpu.VMEM((1, CH), jnp.int32),
            pltpu.VMEM((1, CH), jnp.int32),
            pltpu.VMEM((CH, D), jnp.float32),
            pltpu.VMEM((CH, D), jnp.float32),
            pltpu.VMEM((CTX_PAD,), jnp.int32),
            pltpu.VMEM((CTX_PAD,), jnp.int32),
            pltpu.VMEM((S,), jnp.int32),
            pltpu.VMEM((CTX_PAD // CH, CH), jnp.int32),
            pltpu.VMEM_SHARED((AGG_ROWS, D), jnp.float32),
            pltpu.SemaphoreType.DMA,
            pltpu.SemaphoreType.DMA,
        ],
    )
    def sc_agg(ptr_hbm, vlist_hbm, alist_hbm, vembed_hbm, aembed_hbm,
               vindex_hbm, vsend_hbm, aggv_hbm, agga_hbm,
               ptr_v, idxb, seg_ch, seg_ch1, rows0, rows1, vind_b, i2_b,
               vsend_buf, ctx_seg, agg_sh, gsem, gsem1):
        cid = lax.axis_index("c")
        sid = lax.axis_index("s")
        ebase = sid * EPT

        pltpu.sync_copy(ptr_hbm, ptr_v)

        def _zrow(i, c):
            for j in range(D // 16):
                rows0[i, pl.ds(j * 16, 16)] = jnp.zeros((16,), jnp.float32)
            return c
        lax.fori_loop(0, CH, _zrow, 0)
        for k in range(ROWS_PT // CH):
            pltpu.sync_copy(rows0, agg_sh.at[pl.ds(sid * ROWS_PT + k * CH, CH)])
        plsc.subcore_barrier()

        def _edge_loop(list_hbm, table_hbm):
            bufs = ((seg_ch, rows0, gsem), (seg_ch1, rows1, gsem1))

            def start(k, b, buf):
                sg, rw, sem = buf
                _seg_search(ptr_v, sg, ebase, b * BLK + k * CH)
                return pltpu.async_copy(
                    table_hbm.at[idxb.at[pl.ds(k * CH, CH)]], rw, sem)

            def finish(hdl, buf):
                sg, rw, _ = buf
                hdl.wait()
                pltpu.sync_copy(rw, agg_sh.at[sg.at[0]], add=True)

            def blk(b, carry):
                pltpu.sync_copy(list_hbm.at[pl.ds(ebase + b * BLK, BLK)], idxb)
                h0 = start(0, b, bufs[0])

                def pair(p, c2):
                    ha = pltpu.make_async_copy(
                        table_hbm.at[pl.ds(0, CH)], rows0, gsem)
                    hb = start(2 * p + 1, b, bufs[1])
                    finish(ha, bufs[0])
                    start(2 * p + 2, b, bufs[0])
                    finish(hb, bufs[1])
                    return c2
                lax.fori_loop(0, (NCH - 1) // 2, pair, carry)
                hz = pltpu.make_async_copy(
                    table_hbm.at[pl.ds(0, CH)], rows0, gsem)
                finish(hz, bufs[0])
                return carry
            lax.fori_loop(0, NBLK, blk, 0)

        @pl.when(cid == 0)
        def _():
            _edge_loop(vlist_hbm, vembed_hbm)

        @pl.when(cid == 1)
        def _():
            _edge_loop(alist_hbm, aembed_hbm)

        @pl.when(cid == 0)
        def _():
            pltpu.sync_copy(vsend_hbm, vsend_buf)
            pltpu.sync_copy(vindex_hbm.at[pl.ds(sid * CTX_STRIDE, CTX_LOAD)],
                            vind_b.at[pl.ds(0, CTX_LOAD)])
            for j in range((CTX_PAD - CTX_LOAD) // 16):
                vind_b[pl.ds(CTX_LOAD + j * 16, 16)] = jnp.zeros((16,), jnp.int32)
            cnt = jnp.where(sid == NTILES - 1, CTX_LOAD, CTX_STRIDE)
            for j in range(CTX_PAD // 16):
                l = j * 16 + lax.broadcasted_iota(jnp.int32, (16,), 0)
                msk = l < cnt
                vi = plsc.load_gather(vind_b, [jnp.where(msk, l, 0)])
                i2 = plsc.load_gather(vsend_buf, [vi])
                i2_b[pl.ds(j * 16, 16)] = i2
                jc, jr = divmod(j * 16, CH)
                ctx_seg[jc, pl.ds(jr, 16)] = jnp.where(msk, CTX_ROW, DUMP_ROW)
            for k in range(CTX_PAD // CH):
                pltpu.async_copy(vembed_hbm.at[i2_b.at[pl.ds(k * CH, CH)]],
                                 rows0, gsem).wait()
                pltpu.sync_copy(rows0, agg_sh.at[ctx_seg.at[k]], add=True)

        plsc.subcore_barrier()

        @pl.when(cid == 0)
        def _():
            pltpu.sync_copy(agg_sh.at[pl.ds(sid * ROWS_PT, ROWS_PT)],
                            aggv_hbm.at[pl.ds(sid * ROWS_PT, ROWS_PT)])

        @pl.when(cid == 1)
        def _():
            pltpu.sync_copy(agg_sh.at[pl.ds(sid * ROWS_PT, ROWS_PT)],
                            agga_hbm.at[pl.ds(sid * ROWS_PT, ROWS_PT)])

    return sc_agg


_sc_agg = _sc_agg_build()


def _mmT(a, b):
    return lax.dot_general(a, b, (((1,), (1,)), ((), ())),
                           precision=lax.Precision.HIGHEST,
                           preferred_element_type=jnp.float32)


def _mm(a, b):
    return lax.dot_general(a, b, (((1,), (0,)), ((), ())),
                           precision=lax.Precision.HIGHEST,
                           preferred_element_type=jnp.float32)


def _tc_body(t_ref, aggv_ref, agga_ref, wv_ref, wav_ref, w1_ref, w11_ref,
             w12_ref, out_ref):
    w1a = w1_ref[:, :D]
    w1b = w1_ref[:, D:]
    b1 = _mm(w1a, w11_ref[...])
    w1b12 = _mm(w1b, w12_ref[...])
    b2 = _mm(w1b12, wv_ref[...])
    b3 = _mm(w1b12, wav_ref[...])
    b4 = _mm(w1b, wv_ref[...])
    ctx = aggv_ref[CTX_ROW:CTX_ROW + 1, :] * (1.0 / R)
    ctxc = _mmT(ctx, b4)
    nb, bs = 10, N_T // 10
    for i in range(nb):
        sl = pl.ds(i * bs, bs)
        out_ref[sl, :] = (_mmT(t_ref[sl, :], b1) + _mmT(aggv_ref[sl, :], b2)
                          + _mmT(agga_ref[sl, :], b3) + ctxc)


def kernel(ptr_t, a_list, v_list, t_embed, v_embed, a_embed, v_index,
           v_id_send, wv, wa_v, w1, w11, w12):
    aggv, agga = _sc_agg(ptr_t, v_list, a_list, v_embed, a_embed,
                         v_index, v_id_send)
    out = pl.pallas_call(
        _tc_body,
        out_shape=jax.ShapeDtypeStruct((N_T, D), jnp.float32),
    )(t_embed, aggv, agga, wv, wa_v, w1, w11, w12)
    return out


# speedup vs baseline: 110.4571x; 1.1455x over previous
import functools

import jax
import jax.numpy as jnp
from jax import lax
from jax.experimental import pallas as pl
from jax.experimental.pallas import tpu as pltpu
from jax.experimental.pallas import tpu_sc as plsc

N_T = 10000
N_V = 10000
E = 320000
D = 128
R = 5000
S = 5000
NPTR = N_T + 1

NTILES = 16
EPT = E // NTILES
BLK = 2000
NBLK = EPT // BLK
CH = 80
NCH = BLK // CH

ROWS_PT = 632
AGG_ROWS = NTILES * ROWS_PT
CTX_ROW = N_T
DUMP_ROW = N_T + 8

CTX_STRIDE = 312
CTX_LOAD = 320
CTX_PAD = 400


def _seg_search(ptr_v, seg_ref, ebase, chunk_base):
    for j in range(CH // 16):
        eg = ebase + chunk_base + j * 16 + lax.broadcasted_iota(jnp.int32, (16,), 0)
        lo = jnp.zeros((16,), jnp.int32)
        hi = jnp.full((16,), NPTR, jnp.int32)
        for _ in range(14):
            mid = jnp.right_shift(lo + hi, 1)
            pv = plsc.load_gather(ptr_v, [mid])
            cond = pv <= eg
            lo = jnp.where(cond, mid + 1, lo)
            hi = jnp.where(cond, hi, mid)
        seg_ref[0, pl.ds(j * 16, 16)] = jnp.clip(lo - 1, 0, N_T - 1)


def _sc_agg_build():
    mesh = plsc.VectorSubcoreMesh(core_axis_name="c", subcore_axis_name="s")

    @functools.partial(
        pl.kernel,
        mesh=mesh,
        compiler_params=pltpu.CompilerParams(needs_layout_passes=False),
        out_type=[
            jax.ShapeDtypeStruct((AGG_ROWS, D), jnp.float32),
            jax.ShapeDtypeStruct((AGG_ROWS, D), jnp.float32),
        ],
        scratch_types=[
            pltpu.VMEM((NPTR,), jnp.int32),
            pltpu.VMEM((BLK,), jnp.int32),
            pltpu.VMEM((1, CH), jnp.int32),
            pltpu.VMEM((1, CH), jnp.int32),
            pltpu.VMEM((1, CH), jnp.int32),
            pltpu.VMEM((CH, D), jnp.float32),
            pltpu.VMEM((CH, D), jnp.float32),
            pltpu.VMEM((CH, D), jnp.float32),
            pltpu.VMEM((S,), jnp.int32),
            pltpu.VMEM((CTX_PAD // CH, CH), jnp.int32),
            pltpu.VMEM_SHARED((AGG_ROWS, D), jnp.float32),
            pltpu.SemaphoreType.DMA,
            pltpu.SemaphoreType.DMA,
            pltpu.SemaphoreType.DMA,
            pltpu.SemaphoreType.DMA,
            pltpu.SemaphoreType.DMA,
            pltpu.SemaphoreType.DMA,
        ],
    )
    def sc_agg(ptr_hbm, vlist_hbm, alist_hbm, vembed_hbm, aembed_hbm,
               vindex_hbm, vsend_hbm, aggv_hbm, agga_hbm,
               ptr_v, idxb, seg_ch, seg_ch1, seg_ch2, rows0, rows1, rows2,
               vsend_buf, ctx_seg, agg_sh,
               gsem, gsem1, gsem2, ssem, ssem1, ssem2):
        cid = lax.axis_index("c")
        sid = lax.axis_index("s")
        ebase = sid * EPT

        pltpu.sync_copy(ptr_hbm, ptr_v)

        def _zrow(i, c):
            for j in range(D // 16):
                rows0[i, pl.ds(j * 16, 16)] = jnp.zeros((16,), jnp.float32)
            return c
        lax.fori_loop(0, CH, _zrow, 0)
        for k in range(ROWS_PT // CH):
            pltpu.sync_copy(rows0, agg_sh.at[pl.ds(sid * ROWS_PT + k * CH, CH)])
        if ROWS_PT % CH:
            pltpu.sync_copy(
                rows0.at[pl.ds(0, ROWS_PT % CH)],
                agg_sh.at[pl.ds(sid * ROWS_PT + (ROWS_PT // CH) * CH,
                                ROWS_PT % CH)])
        plsc.subcore_barrier()

        def _edge_loop(list_hbm, table_hbm):
            segb = (seg_ch, seg_ch1, seg_ch2)
            rowsb = (rows0, rows1, rows2)
            gsems = (gsem, gsem1, gsem2)
            ssems = (ssem, ssem1, ssem2)

            def g_start(k, b, q):
                _seg_search(ptr_v, segb[q], ebase, b * BLK + k * CH)
                pltpu.async_copy(table_hbm.at[idxb.at[pl.ds(k * CH, CH)]],
                                 rowsb[q], gsems[q])

            def g_wait(q):
                pltpu.make_async_copy(table_hbm.at[pl.ds(0, CH)], rowsb[q],
                                      gsems[q]).wait()

            def s_start(q):
                pltpu.async_copy(rowsb[q], agg_sh.at[segb[q].at[0]], ssems[q],
                                 add=True)

            def s_wait(q):
                pltpu.make_async_copy(rowsb[q], agg_sh.at[segb[q].at[0]],
                                      ssems[q]).wait()

            def blk(b, carry):
                pltpu.sync_copy(list_hbm.at[pl.ds(ebase + b * BLK, BLK)], idxb)
                g_start(0, b, 0)

                def triple(t, c2):
                    for q in range(3):
                        k = 3 * t + q
                        if q < 2:
                            @pl.when(t > 0)
                            def _():
                                s_wait((q + 1) % 3)
                        else:
                            s_wait((q + 1) % 3)
                        g_start(k + 1, b, (q + 1) % 3)
                        g_wait(q)
                        s_start(q)
                    return c2
                lax.fori_loop(0, (NCH - 1) // 3, triple, carry)
                s_wait(1)
                g_wait(0)
                s_start(0)
                s_wait(2)
                s_wait(0)
                return carry
            lax.fori_loop(0, NBLK, blk, 0)

        @pl.when(cid == 0)
        def _():
            _edge_loop(vlist_hbm, vembed_hbm)

        @pl.when(cid == 1)
        def _():
            _edge_loop(alist_hbm, aembed_hbm)

        @pl.when(cid == 0)
        def _():
            pltpu.sync_copy(vsend_hbm, vsend_buf)
            pltpu.sync_copy(vindex_hbm.at[pl.ds(sid * CTX_STRIDE, CTX_LOAD)],
                            idxb.at[pl.ds(0, CTX_LOAD)])
            for j in range((CTX_PAD - CTX_LOAD) // 16):
                idxb[pl.ds(CTX_LOAD + j * 16, 16)] = jnp.zeros((16,), jnp.int32)
            cnt = jnp.where(sid == NTILES - 1, CTX_LOAD, CTX_STRIDE)
            for j in range(CTX_PAD // 16):
                l = j * 16 + lax.broadcasted_iota(jnp.int32, (16,), 0)
                msk = l < cnt
                vi = plsc.load_gather(idxb, [jnp.where(msk, l, 0)])
                i2 = plsc.load_gather(vsend_buf, [vi])
                idxb[pl.ds(800 + j * 16, 16)] = i2
                jc, jr = divmod(j * 16, CH)
                ctx_seg[jc, pl.ds(jr, 16)] = jnp.where(msk, CTX_ROW, DUMP_ROW)
            for k in range(CTX_PAD // CH):
                pltpu.async_copy(
                    vembed_hbm.at[idxb.at[pl.ds(800 + k * CH, CH)]],
                    rows0, gsem).wait()
                pltpu.sync_copy(rows0, agg_sh.at[ctx_seg.at[k]], add=True)

        plsc.subcore_barrier()

        @pl.when(cid == 0)
        def _():
            pltpu.sync_copy(agg_sh.at[pl.ds(sid * ROWS_PT, ROWS_PT)],
                            aggv_hbm.at[pl.ds(sid * ROWS_PT, ROWS_PT)])

        @pl.when(cid == 1)
        def _():
            pltpu.sync_copy(agg_sh.at[pl.ds(sid * ROWS_PT, ROWS_PT)],
                            agga_hbm.at[pl.ds(sid * ROWS_PT, ROWS_PT)])

    return sc_agg


_sc_agg = _sc_agg_build()


def _mmT(a, b):
    return lax.dot_general(a, b, (((1,), (1,)), ((), ())),
                           precision=lax.Precision.HIGHEST,
                           preferred_element_type=jnp.float32)


def _mm(a, b):
    return lax.dot_general(a, b, (((1,), (0,)), ((), ())),
                           precision=lax.Precision.HIGHEST,
                           preferred_element_type=jnp.float32)


def _tc_body(t_ref, aggv_ref, agga_ref, wv_ref, wav_ref, w1_ref, w11_ref,
             w12_ref, out_ref):
    w1a = w1_ref[:, :D]
    w1b = w1_ref[:, D:]
    b1 = _mm(w1a, w11_ref[...])
    w1b12 = _mm(w1b, w12_ref[...])
    b2 = _mm(w1b12, wv_ref[...])
    b3 = _mm(w1b12, wav_ref[...])
    b4 = _mm(w1b, wv_ref[...])
    ctx = aggv_ref[CTX_ROW:CTX_ROW + 1, :] * (1.0 / R)
    ctxc = _mmT(ctx, b4)
    nb, bs = 10, N_T // 10
    for i in range(nb):
        sl = pl.ds(i * bs, bs)
        out_ref[sl, :] = (_mmT(t_ref[sl, :], b1) + _mmT(aggv_ref[sl, :], b2)
                          + _mmT(agga_ref[sl, :], b3) + ctxc)


def kernel(ptr_t, a_list, v_list, t_embed, v_embed, a_embed, v_index,
           v_id_send, wv, wa_v, w1, w11, w12):
    aggv, agga = _sc_agg(ptr_t, v_list, a_list, v_embed, a_embed,
                         v_index, v_id_send)
    out = pl.pallas_call(
        _tc_body,
        out_shape=jax.ShapeDtypeStruct((N_T, D), jnp.float32),
    )(t_embed, aggv, agga, wv, wa_v, w1, w11, w12)
    return out


# speedup vs baseline: 117.8313x; 1.0668x over previous
import functools

import jax
import jax.numpy as jnp
from jax import lax
from jax.experimental import pallas as pl
from jax.experimental.pallas import tpu as pltpu
from jax.experimental.pallas import tpu_sc as plsc

N_T = 10000
N_V = 10000
E = 320000
D = 128
R = 5000
S = 5000
NPTR = N_T + 1

NTILES = 16
EPT = E // NTILES
BLK = 2000
NBLK = EPT // BLK
CH = 80
NCH = BLK // CH

ROWS_PT = 632
AGG_ROWS = NTILES * ROWS_PT
CTX_ROW = N_T
DUMP_ROW = N_T + 8

CTX_STRIDE = 312
CTX_LOAD = 320
CTX_PAD = 400


def _seg_search(ptr_v, seg_ref, ebase, chunk_base):
    for j in range(CH // 16):
        eg = ebase + chunk_base + j * 16 + lax.broadcasted_iota(jnp.int32, (16,), 0)
        lo = jnp.zeros((16,), jnp.int32)
        hi = jnp.full((16,), NPTR, jnp.int32)
        for _ in range(14):
            mid = jnp.right_shift(lo + hi, 1)
            pv = plsc.load_gather(ptr_v, [mid])
            cond = pv <= eg
            lo = jnp.where(cond, mid + 1, lo)
            hi = jnp.where(cond, hi, mid)
        seg_ref[0, pl.ds(j * 16, 16)] = jnp.clip(lo - 1, 0, N_T - 1)


def _sc_agg_build():
    mesh = plsc.VectorSubcoreMesh(core_axis_name="c", subcore_axis_name="s")

    @functools.partial(
        pl.kernel,
        mesh=mesh,
        compiler_params=pltpu.CompilerParams(needs_layout_passes=False),
        out_type=[
            jax.ShapeDtypeStruct((AGG_ROWS, D), jnp.float32),
            jax.ShapeDtypeStruct((AGG_ROWS, D), jnp.float32),
        ],
        scratch_types=[
            pltpu.VMEM((NPTR,), jnp.int32),
            pltpu.VMEM((BLK,), jnp.int32),
            pltpu.VMEM((1, CH), jnp.int32),
            pltpu.VMEM((1, CH), jnp.int32),
            pltpu.VMEM((1, CH), jnp.int32),
            pltpu.VMEM((CH, D), jnp.float32),
            pltpu.VMEM((CH, D), jnp.float32),
            pltpu.VMEM((CH, D), jnp.float32),
            pltpu.VMEM((S,), jnp.int32),
            pltpu.VMEM((CTX_PAD // CH, CH), jnp.int32),
            pltpu.VMEM_SHARED((AGG_ROWS, D), jnp.float32),
            pltpu.SemaphoreType.DMA,
            pltpu.SemaphoreType.DMA,
            pltpu.SemaphoreType.DMA,
            pltpu.SemaphoreType.DMA,
            pltpu.SemaphoreType.DMA,
            pltpu.SemaphoreType.DMA,
        ],
    )
    def sc_agg(ptr_hbm, vlist_hbm, alist_hbm, vembed_hbm, aembed_hbm,
               vindex_hbm, vsend_hbm, aggv_hbm, agga_hbm,
               ptr_v, idxb, seg_ch, seg_ch1, seg_ch2, rows0, rows1, rows2,
               vsend_buf, ctx_seg, agg_sh,
               gsem, gsem1, gsem2, ssem, ssem1, ssem2):
        cid = lax.axis_index("c")
        sid = lax.axis_index("s")
        ebase = sid * EPT

        pltpu.sync_copy(ptr_hbm, ptr_v)

        def _zrow(i, c):
            for j in range(D // 16):
                rows0[i, pl.ds(j * 16, 16)] = jnp.zeros((16,), jnp.float32)
            return c
        lax.fori_loop(0, CH, _zrow, 0)
        for k in range(ROWS_PT // CH):
            pltpu.sync_copy(rows0, agg_sh.at[pl.ds(sid * ROWS_PT + k * CH, CH)])
        if ROWS_PT % CH:
            pltpu.sync_copy(
                rows0.at[pl.ds(0, ROWS_PT % CH)],
                agg_sh.at[pl.ds(sid * ROWS_PT + (ROWS_PT // CH) * CH,
                                ROWS_PT % CH)])
        plsc.subcore_barrier()

        def _edge_loop(list_hbm, table_hbm):
            segb = (seg_ch, seg_ch1, seg_ch2)
            rowsb = (rows0, rows1, rows2)
            gsems = (gsem, gsem1, gsem2)
            ssems = (ssem, ssem1, ssem2)

            def g_start(k, b, q):
                _seg_search(ptr_v, segb[q], ebase, b * BLK + k * CH)
                pltpu.async_copy(table_hbm.at[idxb.at[pl.ds(k * CH, CH)]],
                                 rowsb[q], gsems[q])

            def g_wait(q):
                pltpu.make_async_copy(table_hbm.at[pl.ds(0, CH)], rowsb[q],
                                      gsems[q]).wait()

            def s_start(q):
                pltpu.async_copy(rowsb[q], agg_sh.at[segb[q].at[0]], ssems[q],
                                 add=True)

            def s_wait(q):
                pltpu.make_async_copy(rowsb[q], agg_sh.at[segb[q].at[0]],
                                      ssems[q]).wait()

            def blk(b, carry):
                pltpu.sync_copy(list_hbm.at[pl.ds(ebase + b * BLK, BLK)], idxb)
                g_start(0, b, 0)

                def triple(t, c2):
                    for q in range(3):
                        k = 3 * t + q
                        if q < 2:
                            @pl.when(t > 0)
                            def _():
                                s_wait((q + 1) % 3)
                        else:
                            s_wait((q + 1) % 3)
                        g_start(k + 1, b, (q + 1) % 3)
                        g_wait(q)
                        s_start(q)
                    return c2
                lax.fori_loop(0, (NCH - 1) // 3, triple, carry)
                s_wait(1)
                g_wait(0)
                s_start(0)
                s_wait(2)
                s_wait(0)
                return carry
            lax.fori_loop(0, NBLK, blk, 0)

        @pl.when(cid == 0)
        def _():
            _edge_loop(vlist_hbm, vembed_hbm)

        @pl.when(cid == 1)
        def _():
            _edge_loop(alist_hbm, aembed_hbm)

        @pl.when(cid == 0)
        def _():
            pltpu.sync_copy(vsend_hbm, vsend_buf)
            pltpu.sync_copy(vindex_hbm.at[pl.ds(sid * CTX_STRIDE, CTX_LOAD)],
                            idxb.at[pl.ds(0, CTX_LOAD)])
            for j in range((CTX_PAD - CTX_LOAD) // 16):
                idxb[pl.ds(CTX_LOAD + j * 16, 16)] = jnp.zeros((16,), jnp.int32)
            cnt = jnp.where(sid == NTILES - 1, CTX_LOAD, CTX_STRIDE)
            for j in range(CTX_PAD // 16):
                l = j * 16 + lax.broadcasted_iota(jnp.int32, (16,), 0)
                msk = l < cnt
                vi = plsc.load_gather(idxb, [jnp.where(msk, l, 0)])
                i2 = plsc.load_gather(vsend_buf, [vi])
                idxb[pl.ds(800 + j * 16, 16)] = i2
                jc, jr = divmod(j * 16, CH)
                ctx_seg[jc, pl.ds(jr, 16)] = jnp.where(msk, CTX_ROW, DUMP_ROW)
            for k in range(CTX_PAD // CH):
                pltpu.async_copy(
                    vembed_hbm.at[idxb.at[pl.ds(800 + k * CH, CH)]],
                    rows0, gsem).wait()
                pltpu.sync_copy(rows0, agg_sh.at[ctx_seg.at[k]], add=True)

        plsc.subcore_barrier()

        @pl.when(cid == 0)
        def _():
            pltpu.sync_copy(agg_sh.at[pl.ds(sid * ROWS_PT, ROWS_PT)],
                            aggv_hbm.at[pl.ds(sid * ROWS_PT, ROWS_PT)])

        @pl.when(cid == 1)
        def _():
            pltpu.sync_copy(agg_sh.at[pl.ds(sid * ROWS_PT, ROWS_PT)],
                            agga_hbm.at[pl.ds(sid * ROWS_PT, ROWS_PT)])

    return sc_agg


_sc_agg = _sc_agg_build()


def _mmT(a, b):
    return lax.dot_general(a, b, (((1,), (1,)), ((), ())),
                           preferred_element_type=jnp.float32)


def _mm(a, b):
    return lax.dot_general(a, b, (((1,), (0,)), ((), ())),
                           preferred_element_type=jnp.float32)


def _tc_body(t_ref, aggv_ref, agga_ref, wv_ref, wav_ref, w1_ref, w11_ref,
             w12_ref, out_ref):
    w1a = w1_ref[:, :D]
    w1b = w1_ref[:, D:]
    b1 = _mm(w1a, w11_ref[...])
    w1b12 = _mm(w1b, w12_ref[...])
    b2 = _mm(w1b12, wv_ref[...])
    b3 = _mm(w1b12, wav_ref[...])
    b4 = _mm(w1b, wv_ref[...])
    ctx = aggv_ref[CTX_ROW:CTX_ROW + 1, :] * (1.0 / R)
    ctxc = _mmT(ctx, b4)
    nb, bs = 10, N_T // 10
    for i in range(nb):
        sl = pl.ds(i * bs, bs)
        out_ref[sl, :] = (_mmT(t_ref[sl, :], b1) + _mmT(aggv_ref[sl, :], b2)
                          + _mmT(agga_ref[sl, :], b3) + ctxc)


def kernel(ptr_t, a_list, v_list, t_embed, v_embed, a_embed, v_index,
           v_id_send, wv, wa_v, w1, w11, w12):
    aggv, agga = _sc_agg(ptr_t, v_list, a_list, v_embed, a_embed,
                         v_index, v_id_send)
    out = pl.pallas_call(
        _tc_body,
        out_shape=jax.ShapeDtypeStruct((N_T, D), jnp.float32),
    )(t_embed, aggv, agga, wv, wa_v, w1, w11, w12)
    return out
